# Initial kernel scaffold; baseline (speedup 1.0000x reference)
import jax
import jax.numpy as jnp
from jax.experimental import pallas as pl


def kernel(x, M, conv_w, conv_b, bn_gamma, bn_beta):
    raise NotImplementedError("write your pallas kernel here")


# speedup vs baseline: 1.1210x; 1.1210x over previous
import functools

import jax
import jax.numpy as jnp
import numpy as np
from jax.experimental import pallas as pl
from jax.experimental.pallas import tpu as pltpu

IN_CH = 96
K_IM = 48
H = 224
W = 224
CB = 8
NC = IN_CH // CB
KS = 7
PAD = 3
EPS = 1e-5


def _sel_scalar(mrow, jidx, mc, c):
    beats = (mrow > mc) | ((mrow == mc) & (jidx < c))
    r = jnp.sum(beats.astype(jnp.int32))
    return jnp.where(r < K_IM, 1.0, 0.0).astype(jnp.float32)


def _reduce_kernel(x_ref, m_ref, ms_ref, out_ref):
    b = pl.program_id(0)
    nc = pl.program_id(1)
    c0 = nc * CB
    mrow = m_ref[0]
    jidx = jax.lax.broadcasted_iota(jnp.int32, (1, IN_CH), 1)

    s_im = mx_im = s_sub = mx_sub = None
    for i in range(CB):
        c = c0 + i
        si = _sel_scalar(mrow, jidx, ms_ref[b, c], c)
        xi = x_ref[0, i]
        mi = xi * si
        ms = xi - mi
        if i == 0:
            s_im, mx_im, s_sub, mx_sub = mi, mi, ms, ms
        else:
            s_im = s_im + mi
            mx_im = jnp.maximum(mx_im, mi)
            s_sub = s_sub + ms
            mx_sub = jnp.maximum(mx_sub, ms)

    @pl.when(nc == 0)
    def _init():
        out_ref[0, 0] = s_im
        out_ref[0, 1] = mx_im
        out_ref[0, 2] = s_sub
        out_ref[0, 3] = mx_sub

    @pl.when(nc > 0)
    def _accum():
        out_ref[0, 0] += s_im
        out_ref[0, 1] = jnp.maximum(out_ref[0, 1], mx_im)
        out_ref[0, 2] += s_sub
        out_ref[0, 3] = jnp.maximum(out_ref[0, 3], mx_sub)

    @pl.when(nc == NC - 1)
    def _finalize():
        out_ref[0, 0] = out_ref[0, 0] * (1.0 / K_IM)
        out_ref[0, 2] = out_ref[0, 2] * (1.0 / (IN_CH - K_IM))


def _conv_kernel(p_ref, w_ref, b_ref, g_ref, bt_ref, out_ref):
    scale = g_ref[0, 0] * np.float32(1.0 / np.sqrt(1.0 + EPS))
    bias = b_ref[0, 0]
    beta = bt_ref[0, 0]
    for half in range(2):
        acc = jnp.zeros((H, W), jnp.float32)
        for ci in range(2):
            src = p_ref[0, 2 * half + ci]
            for kh in range(KS):
                for kw in range(KS):
                    acc += w_ref[0, ci, kh, kw] * src[kh:kh + H, kw:kw + W]
        h = (acc + bias) * scale + beta
        out_ref[0, half] = jax.nn.sigmoid(jax.nn.relu(h))


def _apply_kernel(x_ref, m_ref, ms_ref, att_ref, oim_ref, osub_ref):
    b = pl.program_id(0)
    nc = pl.program_id(1)
    c0 = nc * CB
    mrow = m_ref[0]
    jidx = jax.lax.broadcasted_iota(jnp.int32, (1, IN_CH), 1)
    att_im = att_ref[0, 0]
    att_sub = att_ref[0, 1]
    for i in range(CB):
        si = _sel_scalar(mrow, jidx, ms_ref[b, c0 + i], c0 + i)
        xi = x_ref[0, i]
        mi = xi * si
        oim_ref[0, i] = mi * att_im
        osub_ref[0, i] = (xi - mi) * att_sub


@jax.jit
def kernel(x, M, conv_w, conv_b, bn_gamma, bn_beta):
    B = x.shape[0]
    m2 = M.reshape(B, 1, IN_CH)
    msc = M.reshape(B, IN_CH)
    f32 = jnp.float32

    maps = pl.pallas_call(
        _reduce_kernel,
        grid=(B, NC),
        in_specs=[
            pl.BlockSpec((1, CB, H, W), lambda b, n: (b, n, 0, 0)),
            pl.BlockSpec((1, 1, IN_CH), lambda b, n: (b, 0, 0)),
            pl.BlockSpec(memory_space=pltpu.SMEM),
        ],
        out_specs=pl.BlockSpec((1, 4, H, W), lambda b, n: (b, 0, 0, 0)),
        out_shape=jax.ShapeDtypeStruct((B, 4, H, W), f32),
    )(x, m2, msc)

    maps_p = jnp.pad(maps, ((0, 0), (0, 0), (PAD, PAD), (PAD, PAD)))

    att = pl.pallas_call(
        _conv_kernel,
        grid=(B,),
        in_specs=[
            pl.BlockSpec((1, 4, H + 2 * PAD, W + 2 * PAD), lambda b: (b, 0, 0, 0)),
            pl.BlockSpec((1, 2, KS, KS), lambda b: (0, 0, 0, 0)),
            pl.BlockSpec((1, 1), lambda b: (0, 0)),
            pl.BlockSpec((1, 1), lambda b: (0, 0)),
            pl.BlockSpec((1, 1), lambda b: (0, 0)),
        ],
        out_specs=pl.BlockSpec((1, 2, H, W), lambda b: (b, 0, 0, 0)),
        out_shape=jax.ShapeDtypeStruct((B, 2, H, W), f32),
    )(maps_p, conv_w, conv_b.reshape(1, 1), bn_gamma.reshape(1, 1),
      bn_beta.reshape(1, 1))

    out_im, out_sub = pl.pallas_call(
        _apply_kernel,
        grid=(B, NC),
        in_specs=[
            pl.BlockSpec((1, CB, H, W), lambda b, n: (b, n, 0, 0)),
            pl.BlockSpec((1, 1, IN_CH), lambda b, n: (b, 0, 0)),
            pl.BlockSpec(memory_space=pltpu.SMEM),
            pl.BlockSpec((1, 2, H, W), lambda b, n: (b, 0, 0, 0)),
        ],
        out_specs=[
            pl.BlockSpec((1, CB, H, W), lambda b, n: (b, n, 0, 0)),
            pl.BlockSpec((1, CB, H, W), lambda b, n: (b, n, 0, 0)),
        ],
        out_shape=[
            jax.ShapeDtypeStruct((B, IN_CH, H, W), f32),
            jax.ShapeDtypeStruct((B, IN_CH, H, W), f32),
        ],
    )(x, m2, msc, att)

    return (out_im, out_sub)


# speedup vs baseline: 1.5253x; 1.3607x over previous
import functools

import jax
import jax.numpy as jnp
import numpy as np
from jax.experimental import pallas as pl
from jax.experimental.pallas import tpu as pltpu

IN_CH = 96
K_IM = 48
H = 224
W = 224
CB = 8
NC = IN_CH // CB
KS = 7
PAD = 3
EPS = 1e-5


def _sel_scalar(mrow, jidx, mc, c):
    beats = (mrow > mc) | ((mrow == mc) & (jidx < c))
    r = jnp.sum(beats.astype(jnp.int32))
    return jnp.where(r < K_IM, 1.0, 0.0).astype(jnp.float32)


def _reduce_kernel(x_ref, m_ref, ms_ref, out_ref):
    b = pl.program_id(0)
    nc = pl.program_id(1)
    c0 = nc * CB
    mrow = m_ref[0]
    jidx = jax.lax.broadcasted_iota(jnp.int32, (1, IN_CH), 1)

    s_im = mx_im = s_sub = mx_sub = None
    for i in range(CB):
        c = c0 + i
        si = _sel_scalar(mrow, jidx, ms_ref[b, c], c)
        xi = x_ref[0, i]
        mi = xi * si
        ms = xi - mi
        if i == 0:
            s_im, mx_im, s_sub, mx_sub = mi, mi, ms, ms
        else:
            s_im = s_im + mi
            mx_im = jnp.maximum(mx_im, mi)
            s_sub = s_sub + ms
            mx_sub = jnp.maximum(mx_sub, ms)

    @pl.when(nc == 0)
    def _init():
        out_ref[0, 0] = s_im
        out_ref[0, 1] = mx_im
        out_ref[0, 2] = s_sub
        out_ref[0, 3] = mx_sub

    @pl.when(nc > 0)
    def _accum():
        out_ref[0, 0] += s_im
        out_ref[0, 1] = jnp.maximum(out_ref[0, 1], mx_im)
        out_ref[0, 2] += s_sub
        out_ref[0, 3] = jnp.maximum(out_ref[0, 3], mx_sub)

    @pl.when(nc == NC - 1)
    def _finalize():
        out_ref[0, 0] = out_ref[0, 0] * (1.0 / K_IM)
        out_ref[0, 2] = out_ref[0, 2] * (1.0 / (IN_CH - K_IM))


def _conv_kernel(p_ref, w_ref, b_ref, g_ref, bt_ref, out_ref, cs_ref):
    for m in range(4):
        for kw in range(KS):
            cs_ref[m, kw] = p_ref[0, m, :, kw:kw + W]
    scale = g_ref[0, 0] * np.float32(1.0 / np.sqrt(1.0 + EPS))
    bias = b_ref[0, 0]
    beta = bt_ref[0, 0]
    for half in range(2):
        acc = jnp.zeros((H, W), jnp.float32)
        for ci in range(2):
            for kh in range(KS):
                for kw in range(KS):
                    acc += w_ref[0, ci, kh, kw] * cs_ref[2 * half + ci, kw,
                                                         kh:kh + H, :]
        h = (acc + bias) * scale + beta
        out_ref[0, half] = jax.nn.sigmoid(jax.nn.relu(h))


def _apply_kernel(x_ref, m_ref, ms_ref, att_ref, oim_ref, osub_ref):
    b = pl.program_id(0)
    nc = pl.program_id(1)
    c0 = nc * CB
    mrow = m_ref[0]
    jidx = jax.lax.broadcasted_iota(jnp.int32, (1, IN_CH), 1)
    att_im = att_ref[0, 0]
    att_sub = att_ref[0, 1]
    for i in range(CB):
        si = _sel_scalar(mrow, jidx, ms_ref[b, c0 + i], c0 + i)
        xi = x_ref[0, i]
        mi = xi * si
        oim_ref[0, i] = mi * att_im
        osub_ref[0, i] = (xi - mi) * att_sub


@jax.jit
def kernel(x, M, conv_w, conv_b, bn_gamma, bn_beta):
    B = x.shape[0]
    m2 = M.reshape(B, 1, IN_CH)
    msc = M.reshape(B, IN_CH)
    f32 = jnp.float32

    maps = pl.pallas_call(
        _reduce_kernel,
        grid=(B, NC),
        in_specs=[
            pl.BlockSpec((1, CB, H, W), lambda b, n: (b, n, 0, 0)),
            pl.BlockSpec((1, 1, IN_CH), lambda b, n: (b, 0, 0)),
            pl.BlockSpec(memory_space=pltpu.SMEM),
        ],
        out_specs=pl.BlockSpec((1, 4, H, W), lambda b, n: (b, 0, 0, 0)),
        out_shape=jax.ShapeDtypeStruct((B, 4, H, W), f32),
    )(x, m2, msc)

    maps_p = jnp.pad(maps, ((0, 0), (0, 0), (PAD, PAD), (PAD, PAD)))

    att = pl.pallas_call(
        _conv_kernel,
        grid=(B,),
        in_specs=[
            pl.BlockSpec((1, 4, H + 2 * PAD, W + 2 * PAD), lambda b: (b, 0, 0, 0)),
            pl.BlockSpec((1, 2, KS, KS), lambda b: (0, 0, 0, 0)),
            pl.BlockSpec((1, 1), lambda b: (0, 0)),
            pl.BlockSpec((1, 1), lambda b: (0, 0)),
            pl.BlockSpec((1, 1), lambda b: (0, 0)),
        ],
        out_specs=pl.BlockSpec((1, 2, H, W), lambda b: (b, 0, 0, 0)),
        out_shape=jax.ShapeDtypeStruct((B, 2, H, W), f32),
        scratch_shapes=[pltpu.VMEM((4, KS, H + 2 * PAD, W), jnp.float32)],
    )(maps_p, conv_w, conv_b.reshape(1, 1), bn_gamma.reshape(1, 1),
      bn_beta.reshape(1, 1))

    out_im, out_sub = pl.pallas_call(
        _apply_kernel,
        grid=(B, NC),
        in_specs=[
            pl.BlockSpec((1, CB, H, W), lambda b, n: (b, n, 0, 0)),
            pl.BlockSpec((1, 1, IN_CH), lambda b, n: (b, 0, 0)),
            pl.BlockSpec(memory_space=pltpu.SMEM),
            pl.BlockSpec((1, 2, H, W), lambda b, n: (b, 0, 0, 0)),
        ],
        out_specs=[
            pl.BlockSpec((1, CB, H, W), lambda b, n: (b, n, 0, 0)),
            pl.BlockSpec((1, CB, H, W), lambda b, n: (b, n, 0, 0)),
        ],
        out_shape=[
            jax.ShapeDtypeStruct((B, IN_CH, H, W), f32),
            jax.ShapeDtypeStruct((B, IN_CH, H, W), f32),
        ],
    )(x, m2, msc, att)

    return (out_im, out_sub)
